# baseline (speedup 1.0000x reference)
import jax
import jax.numpy as jnp
from jax import lax
from jax.experimental import pallas as pl
from jax.experimental.pallas import tpu as pltpu
from jax.experimental.pallas import tpu_sc as plsc

NSEG = 1024
BLK = 2048
_N_CONST = 100000

NC, NS, L = 2, 16, 16
NW = NC * NS
T = 128
FULL_TILES = _N_CONST // T
TAIL = _N_CONST - FULL_TILES * T
TAIL_WID = FULL_TILES % NW
ITERS = FULL_TILES // NW + 1


def _make_gate_body(row_off):
  def _gate_body(x_ref, w1_ref, b1_ref, w2_ref, b2_ref, gate_ref, m_ref, macc):
    b = pl.program_id(0)
    nb = pl.num_programs(0)
    x = x_ref[...]
    h = jnp.tanh(
        jax.lax.dot_general(x, w1_ref[...], (((1,), (0,)), ((), ())),
                            preferred_element_type=jnp.float32)
        + b1_ref[...]
    )
    g = jax.lax.dot_general(w2_ref[...], h, (((0,), (1,)), ((), ())),
                            preferred_element_type=jnp.float32)
    g = g + b2_ref[0]
    col = row_off + b * BLK + jax.lax.broadcasted_iota(jnp.int32, (1, BLK), 1)
    g = jnp.where(col < _N_CONST, g, jnp.float32(-1e30))
    gate_ref[...] = g

    bmax = jnp.max(g)

    @pl.when(b == 0)
    def _init():
        macc[...] = jnp.full_like(macc, -1e30)

    macc[...] = jnp.maximum(macc[...], bmax)

    @pl.when(b == nb - 1)
    def _fin():
        m_ref[...] = macc[...]

  return _gate_body


def _gate_pass(x, W1, b1, W2, b2, blk_lo, nb):
    gate, m = pl.pallas_call(
        _make_gate_body(blk_lo * BLK),
        grid=(nb,),
        in_specs=[
            pl.BlockSpec((BLK, 128), lambda b: (b + blk_lo, 0)),
            pl.BlockSpec((128, 64), lambda b: (0, 0)),
            pl.BlockSpec((64,), lambda b: (0,)),
            pl.BlockSpec((64, 1), lambda b: (0, 0)),
            pl.BlockSpec((1,), lambda b: (0,)),
        ],
        out_specs=[
            pl.BlockSpec((1, BLK), lambda b: (0, b)),
            pl.BlockSpec((1, 128), lambda b: (0, 0)),
        ],
        out_shape=[
            jax.ShapeDtypeStruct((1, nb * BLK), jnp.float32),
            jax.ShapeDtypeStruct((1, 128), jnp.float32),
        ],
        scratch_shapes=[pltpu.VMEM((1, 128), jnp.float32)],
    )(x, W1, b1, W2, b2)
    return gate.reshape(nb * BLK), m


def _combine_body(*refs):
    out_ref = refs[-1]
    parts = [refs[i:i + 3] for i in range(0, len(refs) - 1, 3)]
    ms = [p[2][...] for p in parts]
    mx = ms[0]
    for mv in ms[1:]:
        mx = jnp.maximum(mx, mv)
    num = jnp.zeros((NSEG, 128), jnp.float32)
    den = jnp.zeros((NSEG, 1), jnp.float32)
    for (num_ref, den_ref, _), mv in zip(parts, ms):
        sc = jnp.exp(mv - mx)
        num = num + sc * (num_ref[0] + num_ref[1])
        den = den + sc * (den_ref[0, :, 0:1] + den_ref[1, :, 0:1])
    out_ref[...] = num / (den + 1e-16)


def _combine(parts):
    flat = [r for part in parts for r in part]
    return pl.pallas_call(
        _combine_body,
        out_shape=jax.ShapeDtypeStruct((NSEG, 128), jnp.float32),
    )(*flat)


def _make_sc_body(t_begin, t_end, gate_off, iters, has_tail):
  tail_wid = (FULL_TILES - t_begin) % NW

  def _sc_body(gate_hbm, batch_hbm, m_hbm, x_hbm, num_out, den_out,
               rows0, rows1, gv0, gv1, ix0, ix1, evec0, evec1, idx_tail,
               mbuf, zbuf, shared_num, shared_den, semx0, semx1, sems0, sems1):
      cid = lax.axis_index("c")
      sid = lax.axis_index("s")
      wid = sid * NC + cid

      rows = (rows0, rows1)
      gv = (gv0, gv1)
      ix = (ix0, ix1)
      evec = (evec0, evec1)
      semx = (semx0, semx1)
      sems = (sems0, sems1)

      z16 = jnp.zeros((16,), jnp.float32)

      def z64(r, _):
          for c in range(8):
              zbuf[r, pl.ds(16 * c, 16)] = z16
          return 0

      def zev(r, _):
          for c in range(8):
              evec0[r, pl.ds(16 * c, 16)] = z16
              evec1[r, pl.ds(16 * c, 16)] = z16
          return 0

      lax.fori_loop(0, 64, z64, 0)
      lax.fori_loop(0, T, zev, 0)
      pltpu.sync_copy(zbuf, shared_num.at[pl.ds(64 * sid, 64)])
      pltpu.sync_copy(zbuf, shared_den.at[pl.ds(64 * sid, 64)])
      pltpu.sync_copy(m_hbm.at[0], mbuf)
      plsc.subcore_barrier()

      mvec = mbuf[pl.ds(0, 16)]
      lane0 = lax.iota(jnp.int32, 16) == 0

      def issue_in(t, P):
          base = t * T
          pltpu.async_copy(x_hbm.at[pl.ds(base, T)], rows[P], semx[P])
          pltpu.async_copy(gate_hbm.at[pl.ds(base - gate_off, T)],
                           gv[P], semx[P])
          pltpu.async_copy(batch_hbm.at[pl.ds(base, T)], ix[P], semx[P])

      def wait_in(P):
          pltpu.make_async_copy(x_hbm.at[pl.ds(0, T)], rows[P], semx[P]).wait()
          pltpu.make_async_copy(gate_hbm.at[pl.ds(0, T)], gv[P], semx[P]).wait()
          pltpu.make_async_copy(batch_hbm.at[pl.ds(0, T)], ix[P], semx[P]).wait()

      def compute(P, nk):
          def kstep(k, _):
              g = gv[P][pl.ds(16 * k, 16)]
              e = jnp.exp(g - mvec)
              for r2 in range(16):
                  er = e[r2]
                  r = 16 * k + r2
                  evec[P][r, pl.ds(0, 16)] = jnp.where(lane0, er, 0.0)
                  for c in range(8):
                      rows[P][r, pl.ds(16 * c, 16)] = (
                          rows[P][r, pl.ds(16 * c, 16)] * er)
              return 0

          lax.fori_loop(0, nk, kstep, 0)

      def issue_scat(P):
          pltpu.async_copy(rows[P], shared_num.at[ix[P]], sems[P], add=True)
          pltpu.async_copy(evec[P], shared_den.at[ix[P]], sems[P], add=True)

      def wait_scat(P):
          pltpu.make_async_copy(rows[P], shared_num.at[ix[P]], sems[P]).wait()
          pltpu.make_async_copy(evec[P], shared_den.at[ix[P]], sems[P]).wait()

      issue_in(t_begin + wid, 0)

      def body(j, _):
          for P in (0, 1):
              Q = 1 - P

              @pl.when(j % 2 == P)
              def _parity():
                  t = t_begin + wid + NW * j

                  @pl.when(t < t_end)
                  def _proc():
                      @pl.when(j >= 1)
                      def _drain_prev():
                          wait_scat(Q)

                      @pl.when(t + NW < t_end)
                      def _prefetch():
                          issue_in(t + NW, Q)

                      wait_in(P)
                      compute(P, 8)
                      issue_scat(P)

          return 0

      lax.fori_loop(0, iters, body, 0)

      p_last = (iters - 1) % 2
      p_prev = (iters - 2) % 2
      last_t = t_begin + wid + NW * (iters - 1)

      @pl.when(last_t < t_end)
      def _drain_last():
          wait_scat(p_last)

      @pl.when(last_t >= t_end)
      def _drain_prev_only():
          wait_scat(p_prev)

      if has_tail:
          @pl.when(wid == tail_wid)
          def _tail():
              base = FULL_TILES * T
              pltpu.sync_copy(x_hbm.at[pl.ds(base, TAIL)],
                              rows0.at[pl.ds(0, TAIL)])
              pltpu.sync_copy(gate_hbm.at[pl.ds(base - gate_off, TAIL)],
                              gv0.at[pl.ds(0, TAIL)])
              pltpu.sync_copy(batch_hbm.at[pl.ds(base, TAIL)], idx_tail)
              compute(0, TAIL // 16)
              pltpu.sync_copy(rows0.at[pl.ds(0, TAIL)],
                              shared_num.at[idx_tail], add=True)
              pltpu.sync_copy(evec0.at[pl.ds(0, TAIL)],
                              shared_den.at[idx_tail], add=True)

      plsc.subcore_barrier()

      pltpu.sync_copy(shared_num.at[pl.ds(64 * sid, 64)],
                      num_out.at[cid, pl.ds(64 * sid, 64)])
      pltpu.sync_copy(shared_den.at[pl.ds(64 * sid, 64)],
                      den_out.at[cid, pl.ds(64 * sid, 64)])


  return _sc_body


def _sc_middle(gate, batch, m, x, t_begin, t_end, gate_off, has_tail):
    n_own = t_end - t_begin
    iters = (n_own + (1 if has_tail else 0) + NW - 1) // NW
    f = pl.kernel(
        _make_sc_body(t_begin, t_end, gate_off, iters, has_tail),
        mesh=plsc.VectorSubcoreMesh(core_axis_name="c", subcore_axis_name="s"),
        out_type=[
            jax.ShapeDtypeStruct((NC, NSEG, 128), jnp.float32),
            jax.ShapeDtypeStruct((NC, NSEG, 128), jnp.float32),
        ],
        scratch_types=[
            pltpu.VMEM((T, 128), jnp.float32),
            pltpu.VMEM((T, 128), jnp.float32),
            pltpu.VMEM((T,), jnp.float32),
            pltpu.VMEM((T,), jnp.float32),
            pltpu.VMEM((T,), jnp.int32),
            pltpu.VMEM((T,), jnp.int32),
            pltpu.VMEM((T, 128), jnp.float32),
            pltpu.VMEM((T, 128), jnp.float32),
            pltpu.VMEM((TAIL,), jnp.int32),
            pltpu.VMEM((128,), jnp.float32),
            pltpu.VMEM((64, 128), jnp.float32),
            pltpu.VMEM_SHARED((NSEG, 128), jnp.float32),
            pltpu.VMEM_SHARED((NSEG, 128), jnp.float32),
            pltpu.SemaphoreType.DMA,
            pltpu.SemaphoreType.DMA,
            pltpu.SemaphoreType.DMA,
            pltpu.SemaphoreType.DMA,
        ],
    )
    return f(gate, batch, m, x)


NB_TOTAL = (_N_CONST + BLK - 1) // BLK
BOUNDS = (0, 17, 33, NB_TOTAL)


def kernel(x, batch, W1, b1, W2, b2):
    n = x.shape[0]
    assert n == _N_CONST
    batch = batch.astype(jnp.int32)

    parts = []
    for i in range(len(BOUNDS) - 1):
        blo, bhi = BOUNDS[i], BOUNDS[i + 1]
        last = i == len(BOUNDS) - 2
        gate_i, m_i = _gate_pass(x, W1, b1, W2, b2, blo, bhi - blo)
        t_lo = blo * BLK // T
        t_hi = FULL_TILES if last else bhi * BLK // T
        num_i, den_i = _sc_middle(gate_i, batch, m_i, x, t_lo, t_hi,
                                  t_lo * T, last)
        parts.append((num_i, den_i, m_i))
    return _combine(parts)


# speedup vs baseline: 1.0344x; 1.0344x over previous
import jax
import jax.numpy as jnp
from jax import lax
from jax.experimental import pallas as pl
from jax.experimental.pallas import tpu as pltpu
from jax.experimental.pallas import tpu_sc as plsc

NSEG = 1024
BLK = 2048
_N_CONST = 100000

NC, NS, L = 2, 16, 16
NW = NC * NS
T = 128
FULL_TILES = _N_CONST // T
TAIL = _N_CONST - FULL_TILES * T
TAIL_WID = FULL_TILES % NW
ITERS = FULL_TILES // NW + 1


def _make_gate_body(row_off):
  def _gate_body(x_ref, w1_ref, b1_ref, w2_ref, b2_ref, gate_ref, m_ref, macc):
    b = pl.program_id(0)
    nb = pl.num_programs(0)
    x = x_ref[...]
    h = jnp.tanh(
        jax.lax.dot_general(x, w1_ref[...], (((1,), (0,)), ((), ())),
                            preferred_element_type=jnp.float32)
        + b1_ref[...]
    )
    g = jax.lax.dot_general(w2_ref[...], h, (((0,), (1,)), ((), ())),
                            preferred_element_type=jnp.float32)
    g = g + b2_ref[0]
    col = row_off + b * BLK + jax.lax.broadcasted_iota(jnp.int32, (1, BLK), 1)
    g = jnp.where(col < _N_CONST, g, jnp.float32(-1e30))
    gate_ref[...] = g

    bmax = jnp.max(g)

    @pl.when(b == 0)
    def _init():
        macc[...] = jnp.full_like(macc, -1e30)

    macc[...] = jnp.maximum(macc[...], bmax)

    @pl.when(b == nb - 1)
    def _fin():
        m_ref[...] = macc[...]

  return _gate_body


def _gate_pass(x, W1, b1, W2, b2, blk_lo, nb):
    gate, m = pl.pallas_call(
        _make_gate_body(blk_lo * BLK),
        grid=(nb,),
        in_specs=[
            pl.BlockSpec((BLK, 128), lambda b: (b + blk_lo, 0)),
            pl.BlockSpec((128, 64), lambda b: (0, 0)),
            pl.BlockSpec((64,), lambda b: (0,)),
            pl.BlockSpec((64, 1), lambda b: (0, 0)),
            pl.BlockSpec((1,), lambda b: (0,)),
        ],
        out_specs=[
            pl.BlockSpec((1, BLK), lambda b: (0, b)),
            pl.BlockSpec((1, 128), lambda b: (0, 0)),
        ],
        out_shape=[
            jax.ShapeDtypeStruct((1, nb * BLK), jnp.float32),
            jax.ShapeDtypeStruct((1, 128), jnp.float32),
        ],
        scratch_shapes=[pltpu.VMEM((1, 128), jnp.float32)],
    )(x, W1, b1, W2, b2)
    return gate.reshape(nb * BLK), m


def _combine_body(num0_ref, den0_ref, m0_ref, num1_ref, den1_ref, m1_ref,
                  out_ref):
    m0 = m0_ref[...]
    m1 = m1_ref[...]
    mx = jnp.maximum(m0, m1)
    s0 = jnp.exp(m0 - mx)
    s1 = jnp.exp(m1 - mx)
    num = s0 * (num0_ref[0] + num0_ref[1]) + s1 * (num1_ref[0] + num1_ref[1])
    den = (s0 * (den0_ref[0, :, 0:1] + den0_ref[1, :, 0:1])
           + s1 * (den1_ref[0, :, 0:1] + den1_ref[1, :, 0:1]))
    out_ref[...] = num / (den + 1e-16)


def _combine(num_p0, den_p0, m0, num_p1, den_p1, m1):
    return pl.pallas_call(
        _combine_body,
        out_shape=jax.ShapeDtypeStruct((NSEG, 128), jnp.float32),
    )(num_p0, den_p0, m0, num_p1, den_p1, m1)


def _make_sc_body(t_begin, t_end, gate_off, iters, has_tail):
  tail_wid = (FULL_TILES - t_begin) % NW

  def _sc_body(gate_hbm, batch_hbm, m_hbm, x_hbm, num_out, den_out,
               rows0, rows1, gv0, gv1, ix0, ix1, evec0, evec1, idx_tail,
               mbuf, zbuf, shared_num, shared_den, semx0, semx1, sems0, sems1):
      cid = lax.axis_index("c")
      sid = lax.axis_index("s")
      wid = sid * NC + cid

      rows = (rows0, rows1)
      gv = (gv0, gv1)
      ix = (ix0, ix1)
      evec = (evec0, evec1)
      semx = (semx0, semx1)
      sems = (sems0, sems1)

      z16 = jnp.zeros((16,), jnp.float32)

      def z64(r, _):
          for c in range(8):
              zbuf[r, pl.ds(16 * c, 16)] = z16
          return 0

      def zev(r, _):
          for c in range(8):
              evec0[r, pl.ds(16 * c, 16)] = z16
              evec1[r, pl.ds(16 * c, 16)] = z16
          return 0

      lax.fori_loop(0, 64, z64, 0)
      lax.fori_loop(0, T, zev, 0)
      pltpu.sync_copy(zbuf, shared_num.at[pl.ds(64 * sid, 64)])
      pltpu.sync_copy(zbuf, shared_den.at[pl.ds(64 * sid, 64)])
      pltpu.sync_copy(m_hbm.at[0], mbuf)
      plsc.subcore_barrier()

      mvec = mbuf[pl.ds(0, 16)]
      lane0 = lax.iota(jnp.int32, 16) == 0

      def issue_in(t, P):
          base = t * T
          pltpu.async_copy(x_hbm.at[pl.ds(base, T)], rows[P], semx[P])
          pltpu.async_copy(gate_hbm.at[pl.ds(base - gate_off, T)],
                           gv[P], semx[P])
          pltpu.async_copy(batch_hbm.at[pl.ds(base, T)], ix[P], semx[P])

      def wait_in(P):
          pltpu.make_async_copy(x_hbm.at[pl.ds(0, T)], rows[P], semx[P]).wait()
          pltpu.make_async_copy(gate_hbm.at[pl.ds(0, T)], gv[P], semx[P]).wait()
          pltpu.make_async_copy(batch_hbm.at[pl.ds(0, T)], ix[P], semx[P]).wait()

      def compute(P, nk):
          def kstep(k, _):
              g = gv[P][pl.ds(16 * k, 16)]
              e = jnp.exp(g - mvec)
              for r2 in range(16):
                  er = e[r2]
                  r = 16 * k + r2
                  evec[P][r, pl.ds(0, 16)] = jnp.where(lane0, er, 0.0)
                  for c in range(8):
                      rows[P][r, pl.ds(16 * c, 16)] = (
                          rows[P][r, pl.ds(16 * c, 16)] * er)
              return 0

          lax.fori_loop(0, nk, kstep, 0)

      def issue_scat(P):
          pltpu.async_copy(rows[P], shared_num.at[ix[P]], sems[P], add=True)
          pltpu.async_copy(evec[P], shared_den.at[ix[P]], sems[P], add=True)

      def wait_scat(P):
          pltpu.make_async_copy(rows[P], shared_num.at[ix[P]], sems[P]).wait()
          pltpu.make_async_copy(evec[P], shared_den.at[ix[P]], sems[P]).wait()

      issue_in(t_begin + wid, 0)

      def body(j, _):
          for P in (0, 1):
              Q = 1 - P

              @pl.when(j % 2 == P)
              def _parity():
                  t = t_begin + wid + NW * j

                  @pl.when(t < t_end)
                  def _proc():
                      @pl.when(j >= 1)
                      def _drain_prev():
                          wait_scat(Q)

                      @pl.when(t + NW < t_end)
                      def _prefetch():
                          issue_in(t + NW, Q)

                      wait_in(P)
                      compute(P, 8)
                      issue_scat(P)

          return 0

      lax.fori_loop(0, iters, body, 0)

      p_last = (iters - 1) % 2
      p_prev = (iters - 2) % 2
      last_t = t_begin + wid + NW * (iters - 1)

      @pl.when(last_t < t_end)
      def _drain_last():
          wait_scat(p_last)

      @pl.when(last_t >= t_end)
      def _drain_prev_only():
          wait_scat(p_prev)

      if has_tail:
          @pl.when(wid == tail_wid)
          def _tail():
              base = FULL_TILES * T
              pltpu.sync_copy(x_hbm.at[pl.ds(base, TAIL)],
                              rows0.at[pl.ds(0, TAIL)])
              pltpu.sync_copy(gate_hbm.at[pl.ds(base - gate_off, TAIL)],
                              gv0.at[pl.ds(0, TAIL)])
              pltpu.sync_copy(batch_hbm.at[pl.ds(base, TAIL)], idx_tail)
              compute(0, TAIL // 16)
              pltpu.sync_copy(rows0.at[pl.ds(0, TAIL)],
                              shared_num.at[idx_tail], add=True)
              pltpu.sync_copy(evec0.at[pl.ds(0, TAIL)],
                              shared_den.at[idx_tail], add=True)

      plsc.subcore_barrier()

      pltpu.sync_copy(shared_num.at[pl.ds(64 * sid, 64)],
                      num_out.at[cid, pl.ds(64 * sid, 64)])
      pltpu.sync_copy(shared_den.at[pl.ds(64 * sid, 64)],
                      den_out.at[cid, pl.ds(64 * sid, 64)])


  return _sc_body


def _sc_middle(gate, batch, m, x, t_begin, t_end, gate_off, has_tail):
    n_own = t_end - t_begin
    iters = (n_own + (1 if has_tail else 0) + NW - 1) // NW
    f = pl.kernel(
        _make_sc_body(t_begin, t_end, gate_off, iters, has_tail),
        mesh=plsc.VectorSubcoreMesh(core_axis_name="c", subcore_axis_name="s"),
        out_type=[
            jax.ShapeDtypeStruct((NC, NSEG, 128), jnp.float32),
            jax.ShapeDtypeStruct((NC, NSEG, 128), jnp.float32),
        ],
        scratch_types=[
            pltpu.VMEM((T, 128), jnp.float32),
            pltpu.VMEM((T, 128), jnp.float32),
            pltpu.VMEM((T,), jnp.float32),
            pltpu.VMEM((T,), jnp.float32),
            pltpu.VMEM((T,), jnp.int32),
            pltpu.VMEM((T,), jnp.int32),
            pltpu.VMEM((T, 128), jnp.float32),
            pltpu.VMEM((T, 128), jnp.float32),
            pltpu.VMEM((TAIL,), jnp.int32),
            pltpu.VMEM((128,), jnp.float32),
            pltpu.VMEM((64, 128), jnp.float32),
            pltpu.VMEM_SHARED((NSEG, 128), jnp.float32),
            pltpu.VMEM_SHARED((NSEG, 128), jnp.float32),
            pltpu.SemaphoreType.DMA,
            pltpu.SemaphoreType.DMA,
            pltpu.SemaphoreType.DMA,
            pltpu.SemaphoreType.DMA,
        ],
    )
    return f(gate, batch, m, x)


SPLIT_BLK = 25
SPLIT_TILE = SPLIT_BLK * BLK // T
NB_TOTAL = (_N_CONST + BLK - 1) // BLK


def kernel(x, batch, W1, b1, W2, b2):
    n = x.shape[0]
    assert n == _N_CONST
    batch = batch.astype(jnp.int32)

    gate0, m0 = _gate_pass(x, W1, b1, W2, b2, 0, SPLIT_BLK)
    gate1, m1 = _gate_pass(x, W1, b1, W2, b2, SPLIT_BLK, NB_TOTAL - SPLIT_BLK)
    num_p0, den_p0 = _sc_middle(gate0, batch, m0, x, 0, SPLIT_TILE, 0, False)
    num_p1, den_p1 = _sc_middle(gate1, batch, m1, x, SPLIT_TILE, FULL_TILES,
                                SPLIT_TILE * T, True)
    return _combine(num_p0, den_p0, m0, num_p1, den_p1, m1)
